# baseline (speedup 1.0000x reference)
import functools

import jax
import jax.numpy as jnp
from jax import lax
from jax.experimental import pallas as pl
from jax.experimental.pallas import tpu as pltpu
from jax.experimental.pallas import tpu_sc as plsc

_SZ = 4096
_N = _SZ * _SZ
_TEMP = 10.0
_EPS = 1e-20
_LN2 = 0.6931471805599453

_NC = 2
_NS = 16
_NW = _NC * _NS
_PW = _N // _NW
_CH = 8192
_NB = _PW // _CH
_GRP = _CH // 16

_P0 = -2.505614661980078
_P1 = 4.0496168925861165
_P2 = -2.0994022811883073
_P3 = 0.6355111165203404
_P4 = -0.0800108769068122


def _log_f32(x):
    b = lax.bitcast_convert_type(x, jnp.int32)
    e = lax.shift_right_logical(b, 23) - 127
    m = lax.bitcast_convert_type(
        (b & 0x7FFFFF) | 0x3F800000, jnp.float32)
    p = jnp.float32(_P4)
    p = p * m + jnp.float32(_P3)
    p = p * m + jnp.float32(_P2)
    p = p * m + jnp.float32(_P1)
    p = p * m + jnp.float32(_P0)
    return (e.astype(jnp.float32) + p) * jnp.float32(_LN2)


def _gumbel(u):
    w = -_log_f32(u + jnp.float32(_EPS)) + jnp.float32(_EPS)
    return -_log_f32(w)


@functools.partial(
    pl.kernel,
    out_type=jax.ShapeDtypeStruct((_N,), jnp.float32),
    mesh=plsc.VectorSubcoreMesh(core_axis_name="c", subcore_axis_name="s"),
    scratch_types=[
        pltpu.VMEM((2 * _CH,), jnp.float32),
        pltpu.VMEM((2 * _CH,), jnp.float32),
        pltpu.VMEM((_CH,), jnp.float32),
    ],
    compiler_params=pltpu.CompilerParams(needs_layout_passes=False),
)
def _gumbel_sc(gm_hbm, u_hbm, out_hbm, gm_v, u_v, out_v):
    wid = lax.axis_index("s") * _NC + lax.axis_index("c")
    base_out = wid * _PW

    def block(b, carry):
        off_out = base_out + b * _CH
        pltpu.sync_copy(gm_hbm.at[pl.ds(off_out * 2, 2 * _CH)], gm_v)
        pltpu.sync_copy(u_hbm.at[pl.ds(off_out * 2, 2 * _CH)], u_v)

        def grp(g, c2):
            idx0 = lax.iota(jnp.int32, 16) * 2 + g * 32
            idx1 = idx0 + 1
            l0 = plsc.load_gather(gm_v, [idx0])
            l1 = plsc.load_gather(gm_v, [idx1])
            u0 = plsc.load_gather(u_v, [idx0])
            u1 = plsc.load_gather(u_v, [idx1])
            z = ((l1 + _gumbel(u1)) - (l0 + _gumbel(u0))) * jnp.float32(1.0 / _TEMP)
            out_v[pl.ds(g * 16, 16)] = 1.0 / (1.0 + jnp.exp(z))
            return c2

        lax.fori_loop(0, _GRP, grp, 0)
        pltpu.sync_copy(out_v, out_hbm.at[pl.ds(off_out, _CH)])
        return carry

    lax.fori_loop(0, _NB, block, 0)


def kernel(gen_matrix, u):
    gm = gen_matrix.reshape(-1)
    uf = u.reshape(-1)
    out = _gumbel_sc(gm, uf)
    return out.reshape(_SZ, _SZ)


# speedup vs baseline: 22.8369x; 22.8369x over previous
import functools

import jax
import jax.numpy as jnp
from jax import lax
from jax.experimental import pallas as pl
from jax.experimental.pallas import tpu as pltpu
from jax.experimental.pallas import tpu_sc as plsc

_SZ = 4096
_NJB = _SZ // 128
_TEMP = 10.0
_EPS = 1e-20
_LN2 = 0.6931471805599453

_NC = 2
_NS = 16
_NW = _NC * _NS
_RW = _SZ // _NW
_RB = 8
_JBB = 16
_NBLK = (_RW // _RB) * (_NJB // _JBB)
_GRP = _RB * _JBB * 8

_P0 = -2.505614661980078
_P1 = 4.0496168925861165
_P2 = -2.0994022811883073
_P3 = 0.6355111165203404
_P4 = -0.0800108769068122


def _log_f32(x):
    b = lax.bitcast_convert_type(x, jnp.int32)
    e = lax.shift_right_logical(b, 23) - 127
    m = lax.bitcast_convert_type(
        (b & 0x7FFFFF) | 0x3F800000, jnp.float32)
    p = jnp.float32(_P4)
    p = p * m + jnp.float32(_P3)
    p = p * m + jnp.float32(_P2)
    p = p * m + jnp.float32(_P1)
    p = p * m + jnp.float32(_P0)
    return (e.astype(jnp.float32) + p) * jnp.float32(_LN2)


def _gumbel(u):
    w = -_log_f32(u + jnp.float32(_EPS)) + jnp.float32(_EPS)
    return -_log_f32(w)


@functools.partial(
    pl.kernel,
    out_type=jax.ShapeDtypeStruct((_SZ, _SZ), jnp.float32),
    mesh=plsc.VectorSubcoreMesh(core_axis_name="c", subcore_axis_name="s"),
    scratch_types=[
        pltpu.VMEM((_RB, _JBB, 2, 128), jnp.float32),
        pltpu.VMEM((_RB, _JBB, 2, 128), jnp.float32),
        pltpu.VMEM((_RB, _JBB * 128), jnp.float32),
    ],
    compiler_params=pltpu.CompilerParams(needs_layout_passes=False),
)
def _gumbel_sc(gm_hbm, u_hbm, out_hbm, g_v, u_v, o_v):
    wid = lax.axis_index("s") * _NC + lax.axis_index("c")
    row0 = wid * _RW

    def block(b, carry):
        r0 = row0 + (b // (_NJB // _JBB)) * _RB
        jb0 = (b % (_NJB // _JBB)) * _JBB
        pltpu.sync_copy(gm_hbm.at[pl.ds(r0, _RB), pl.ds(jb0, _JBB)], g_v)
        pltpu.sync_copy(u_hbm.at[pl.ds(r0, _RB), pl.ds(jb0, _JBB)], u_v)

        def grp(g, c2):
            r = lax.shift_right_logical(g, 7)
            rem = g & 127
            jb = lax.shift_right_logical(rem, 3)
            jw = (rem & 7) * 16
            l0 = g_v[r, jb, 0, pl.ds(jw, 16)]
            l1 = g_v[r, jb, 1, pl.ds(jw, 16)]
            u0 = u_v[r, jb, 0, pl.ds(jw, 16)]
            u1 = u_v[r, jb, 1, pl.ds(jw, 16)]
            z = ((l1 + _gumbel(u1)) - (l0 + _gumbel(u0))) * jnp.float32(1.0 / _TEMP)
            o_v[r, pl.ds(jb * 128 + jw, 16)] = 1.0 / (1.0 + jnp.exp(z))
            return c2

        lax.fori_loop(0, _GRP, grp, 0)
        pltpu.sync_copy(o_v, out_hbm.at[pl.ds(r0, _RB), pl.ds(jb0 * 128, _JBB * 128)])
        return carry

    lax.fori_loop(0, _NBLK, block, 0)


def kernel(gen_matrix, u):
    gv = gen_matrix.reshape(_SZ, _NJB, 128, 2).transpose(0, 1, 3, 2)
    uv = u.reshape(_SZ, _NJB, 128, 2).transpose(0, 1, 3, 2)
    return _gumbel_sc(gv, uv)


# speedup vs baseline: 29.8243x; 1.3060x over previous
import functools

import jax
import jax.numpy as jnp
from jax import lax
from jax.experimental import pallas as pl
from jax.experimental.pallas import tpu as pltpu
from jax.experimental.pallas import tpu_sc as plsc

_SZ = 4096
_NJB = _SZ // 128
_TEMP = 10.0
_EPS = 1e-20
_LN2 = 0.6931471805599453

_P0 = -2.505614661980078
_P1 = 4.0496168925861165
_P2 = -2.0994022811883073
_P3 = 0.6355111165203404
_P4 = -0.0800108769068122


def _log_f32(x):
    b = lax.bitcast_convert_type(x, jnp.int32)
    e = lax.shift_right_logical(b, 23) - 127
    m = lax.bitcast_convert_type(
        (b & 0x7FFFFF) | 0x3F800000, jnp.float32)
    p = jnp.float32(_P4)
    p = p * m + jnp.float32(_P3)
    p = p * m + jnp.float32(_P2)
    p = p * m + jnp.float32(_P1)
    p = p * m + jnp.float32(_P0)
    return (e.astype(jnp.float32) + p) * jnp.float32(_LN2)


def _gumbel(u):
    w = -_log_f32(u + jnp.float32(_EPS)) + jnp.float32(_EPS)
    return -_log_f32(w)


_RB_TC = 64


def _tc_body(g_ref, u_ref, o_ref):
    for jb in range(_NJB):
        l0 = g_ref[:, jb, 0, :]
        l1 = g_ref[:, jb, 1, :]
        u0 = u_ref[:, jb, 0, :]
        u1 = u_ref[:, jb, 1, :]
        z = ((l1 + _gumbel(u1)) - (l0 + _gumbel(u0))) * jnp.float32(1.0 / _TEMP)
        o_ref[:, jb * 128:(jb + 1) * 128] = 1.0 / (1.0 + jnp.exp(z))


def kernel(gen_matrix, u):
    gv = gen_matrix.reshape(_SZ, _NJB, 128, 2).transpose(0, 1, 3, 2)
    uv = u.reshape(_SZ, _NJB, 128, 2).transpose(0, 1, 3, 2)
    return pl.pallas_call(
        _tc_body,
        out_shape=jax.ShapeDtypeStruct((_SZ, _SZ), jnp.float32),
        grid=(_SZ // _RB_TC,),
        in_specs=[
            pl.BlockSpec((_RB_TC, _NJB, 2, 128), lambda i: (i, 0, 0, 0)),
            pl.BlockSpec((_RB_TC, _NJB, 2, 128), lambda i: (i, 0, 0, 0)),
        ],
        out_specs=pl.BlockSpec((_RB_TC, _SZ), lambda i: (i, 0)),
    )(gv, uv)


# speedup vs baseline: 196.7492x; 6.5969x over previous
import functools

import jax
import jax.numpy as jnp
from jax import lax
from jax.experimental import pallas as pl
from jax.experimental.pallas import tpu as pltpu
from jax.experimental.pallas import tpu_sc as plsc

_SZ = 4096
_NJB = _SZ // 128
_TEMP = 10.0
_EPS = 1e-20
_LN2 = 0.6931471805599453

_P0 = -2.505614661980078
_P1 = 4.0496168925861165
_P2 = -2.0994022811883073
_P3 = 0.6355111165203404
_P4 = -0.0800108769068122


def _log_f32(x):
    b = lax.bitcast_convert_type(x, jnp.int32)
    e = lax.shift_right_logical(b, 23) - 127
    m = lax.bitcast_convert_type(
        (b & 0x7FFFFF) | 0x3F800000, jnp.float32)
    p = jnp.float32(_P4)
    p = p * m + jnp.float32(_P3)
    p = p * m + jnp.float32(_P2)
    p = p * m + jnp.float32(_P1)
    p = p * m + jnp.float32(_P0)
    return (e.astype(jnp.float32) + p) * jnp.float32(_LN2)


def _gumbel(u):
    w = -_log_f32(u + jnp.float32(_EPS)) + jnp.float32(_EPS)
    return -_log_f32(w)


_RB_TC = 64


def _tc_body(g_ref, u_ref, o_ref):
    l0 = g_ref[:, 0::2, :]
    l1 = g_ref[:, 1::2, :]
    u0 = u_ref[:, 0::2, :]
    u1 = u_ref[:, 1::2, :]
    a0 = l0 + _gumbel(u0)
    a1 = l1 + _gumbel(u1)
    z = (a1 - a0) * jnp.float32(1.0 / _TEMP)
    s = 1.0 / (1.0 + jnp.exp(z))
    for jb in range(_NJB):
        o_ref[:, jb * 128:(jb + 1) * 128] = s[:, jb, :]


def kernel(gen_matrix, u):
    gv = gen_matrix.reshape(_SZ, _NJB, 128, 2).transpose(0, 1, 3, 2) \
                   .reshape(_SZ, 2 * _NJB, 128)
    uv = u.reshape(_SZ, _NJB, 128, 2).transpose(0, 1, 3, 2) \
          .reshape(_SZ, 2 * _NJB, 128)
    return pl.pallas_call(
        _tc_body,
        out_shape=jax.ShapeDtypeStruct((_SZ, _SZ), jnp.float32),
        grid=(_SZ // _RB_TC,),
        in_specs=[
            pl.BlockSpec((_RB_TC, 2 * _NJB, 128), lambda i: (i, 0, 0)),
            pl.BlockSpec((_RB_TC, 2 * _NJB, 128), lambda i: (i, 0, 0)),
        ],
        out_specs=pl.BlockSpec((_RB_TC, _SZ), lambda i: (i, 0)),
    )(gv, uv)


# speedup vs baseline: 240.1862x; 1.2208x over previous
import functools

import jax
import jax.numpy as jnp
from jax import lax
from jax.experimental import pallas as pl
from jax.experimental.pallas import tpu as pltpu
from jax.experimental.pallas import tpu_sc as plsc

_SZ = 4096
_NJB = _SZ // 128
_TEMP = 10.0
_EPS = 1e-20
_LN2 = 0.6931471805599453

_A0 = -2.1545013016129446
_A1 = 3.0445241791721527
_A2 = -1.0464089909355754
_A3 = 0.15638611337636774


def _log2_f32(x):
    b = lax.bitcast_convert_type(x, jnp.int32)
    ef = lax.shift_right_logical(b, 23).astype(jnp.float32)
    m = lax.bitcast_convert_type(
        (b & 0x7FFFFF) | 0x3F800000, jnp.float32)
    p = jnp.float32(_A3)
    p = p * m + jnp.float32(_A2)
    p = p * m + jnp.float32(_A1)
    p = p * m + jnp.float32(_A0 - 127.0)
    return ef + p


def _glog2(u):
    y = _log2_f32(u + jnp.float32(_EPS))
    return _log2_f32(jnp.float32(0.0) - y)


_RB_TC = 64


def _tc_body(g_ref, u_ref, o_ref):
    l0 = g_ref[:, 0::2, :]
    l1 = g_ref[:, 1::2, :]
    t0 = _glog2(u_ref[:, 0::2, :])
    t1 = _glog2(u_ref[:, 1::2, :])
    z2 = ((l1 - l0) * jnp.float32(1.0 / (_TEMP * _LN2))
          + (t0 - t1) * jnp.float32(1.0 / _TEMP))
    s = 1.0 / (1.0 + jnp.exp2(z2))
    for jb in range(_NJB):
        o_ref[:, jb * 128:(jb + 1) * 128] = s[:, jb, :]


def kernel(gen_matrix, u):
    gv = gen_matrix.reshape(_SZ, _NJB, 128, 2).transpose(0, 1, 3, 2) \
                   .reshape(_SZ, 2 * _NJB, 128)
    uv = u.reshape(_SZ, _NJB, 128, 2).transpose(0, 1, 3, 2) \
          .reshape(_SZ, 2 * _NJB, 128)
    return pl.pallas_call(
        _tc_body,
        out_shape=jax.ShapeDtypeStruct((_SZ, _SZ), jnp.float32),
        grid=(_SZ // _RB_TC,),
        in_specs=[
            pl.BlockSpec((_RB_TC, 2 * _NJB, 128), lambda i: (i, 0, 0)),
            pl.BlockSpec((_RB_TC, 2 * _NJB, 128), lambda i: (i, 0, 0)),
        ],
        out_specs=pl.BlockSpec((_RB_TC, _SZ), lambda i: (i, 0)),
    )(gv, uv)
